# speedup vs baseline: 11.2733x; 11.2733x over previous
import jax
import jax.numpy as jnp
from jax.experimental import pallas as pl
from jax.experimental.pallas import tpu as pltpu

_W = 128
_BI = 8


def _build_r(embed_ref, r_ref):
    r_ref[0:384, :] = jnp.broadcast_to(embed_ref[256:257, :], (384, _W))
    row = jax.lax.broadcasted_iota(jnp.int32, (256, 256), 0)
    col = jax.lax.broadcasted_iota(jnp.int32, (256, 256), 1)
    anti = (row + col == 255).astype(jnp.float32)
    r_ref[384:640, :] = jnp.dot(
        anti, embed_ref[0:256, :], preferred_element_type=jnp.float32
    )
    r_ref[640:1024, :] = jnp.broadcast_to(embed_ref[0:1, :], (384, _W))


def _body(embed_ref, out_ref, r_ref):
    b = pl.program_id(0)
    ib = pl.program_id(1)

    @pl.when((b == 0) & (ib == 0))
    def _():
        _build_r(embed_ref, r_ref)

    for ii in range(_BI):
        i = ib * _BI + ii
        out_ref[0, ii] = r_ref[pl.ds(511 - i, 512), :]


def kernel(x, embed):
    b, length, _ = x.shape
    d = embed.shape[1]
    return pl.pallas_call(
        _body,
        grid=(b, length // _BI),
        in_specs=[pl.BlockSpec((2 * _W + 1, d), lambda bb, ib: (0, 0))],
        out_specs=pl.BlockSpec((1, _BI, length, d), lambda bb, ib: (bb, ib, 0, 0)),
        out_shape=jax.ShapeDtypeStruct((b, length, length, d), jnp.float32),
        scratch_shapes=[pltpu.VMEM((1024, d), jnp.float32)],
    )(embed)


# speedup vs baseline: 13.2386x; 1.1743x over previous
import jax
import jax.numpy as jnp
from jax.experimental import pallas as pl
from jax.experimental.pallas import tpu as pltpu

_W = 128
_BI = 16


def _build_r(embed_ref, r_ref):
    r_ref[0:384, :] = jnp.broadcast_to(embed_ref[256:257, :], (384, _W))
    row = jax.lax.broadcasted_iota(jnp.int32, (256, 256), 0)
    col = jax.lax.broadcasted_iota(jnp.int32, (256, 256), 1)
    anti = (row + col == 255).astype(jnp.float32)
    r_ref[384:640, :] = jnp.dot(
        anti, embed_ref[0:256, :], preferred_element_type=jnp.float32
    )
    r_ref[640:1024, :] = jnp.broadcast_to(embed_ref[0:1, :], (384, _W))


def _body(embed_ref, out_ref, r_ref):
    b = pl.program_id(0)
    ib = pl.program_id(1)

    @pl.when((b == 0) & (ib == 0))
    def _():
        _build_r(embed_ref, r_ref)

    for ii in range(_BI):
        i = ib * _BI + ii
        out_ref[0, ii] = r_ref[pl.ds(511 - i, 512), :]


def kernel(x, embed):
    b, length, _ = x.shape
    d = embed.shape[1]
    return pl.pallas_call(
        _body,
        grid=(b, length // _BI),
        in_specs=[pl.BlockSpec((2 * _W + 1, d), lambda bb, ib: (0, 0))],
        out_specs=pl.BlockSpec((1, _BI, length, d), lambda bb, ib: (bb, ib, 0, 0)),
        out_shape=jax.ShapeDtypeStruct((b, length, length, d), jnp.float32),
        scratch_shapes=[pltpu.VMEM((1024, d), jnp.float32)],
    )(embed)
